# baseline (speedup 1.0000x reference)
import functools

import jax
import jax.numpy as jnp
from jax import lax
from jax.experimental import pallas as pl
from jax.experimental.pallas import tpu as pltpu
from jax.experimental.pallas import tpu_sc as plsc


def _make_gather(B, N, D):
    info = plsc.get_sparse_core_info()
    NC, NS, L = info.num_cores, info.num_subcores, info.num_lanes
    NW = NC * NS
    assert B % (8 * NW) == 0 and D % L == 0
    b_per_w = B // NW
    mesh = plsc.VectorSubcoreMesh(core_axis_name="c", subcore_axis_name="s")

    @functools.partial(
        pl.kernel,
        mesh=mesh,
        out_type=jax.ShapeDtypeStruct((B, D), jnp.float32),
        scratch_types=[
            pltpu.VMEM((b_per_w,), jnp.int32),
            pltpu.VMEM((b_per_w, D), jnp.float32),
            pltpu.SemaphoreType.DMA,
        ],
        compiler_params=pltpu.CompilerParams(
            use_tc_tiling_on_sc=True
        ),
    )
    def k(inp_hbm, idx_hbm, out_hbm, idx_v, rows_v, sem):
        wid = lax.axis_index("s") * NC + lax.axis_index("c")
        base = wid * b_per_w
        pltpu.sync_copy(idx_hbm.at[pl.ds(base, b_per_w)], idx_v)
        for c0 in range(0, b_per_w, L):
            vec = idx_v[pl.ds(c0, L)]
            handles = []
            for j in range(L):
                n = vec[j]
                handles.append(
                    pltpu.async_copy(
                        inp_hbm.at[base + c0 + j, n], rows_v.at[c0 + j], sem
                    )
                )
            for h in handles:
                h.wait()
        pltpu.sync_copy(rows_v, out_hbm.at[pl.ds(base, b_per_w)])

    return k


def kernel(inputs, indexs):
    B, N, D = inputs.shape
    idx = indexs.reshape(B)
    return _make_gather(B, N, D)(inputs, idx)


# speedup vs baseline: 3.7195x; 3.7195x over previous
import functools

import jax
import jax.numpy as jnp
from jax import lax
from jax.experimental import pallas as pl
from jax.experimental.pallas import tpu as pltpu
from jax.experimental.pallas import tpu_sc as plsc


def _make_gather(B, N, D):
    info = plsc.get_sparse_core_info()
    NC, NS, L = info.num_cores, info.num_subcores, info.num_lanes
    NW = NC * NS
    assert B % (8 * NW) == 0 and D % L == 0
    b_per_w = B // NW
    CH = 4
    NBUF = 2
    n_chunks = b_per_w // CH
    mesh = plsc.VectorSubcoreMesh(core_axis_name="c", subcore_axis_name="s")

    @functools.partial(
        pl.kernel,
        mesh=mesh,
        out_type=jax.ShapeDtypeStruct((D, B), jnp.float32),
        scratch_types=[
            pltpu.VMEM((b_per_w,), jnp.int32),
            pltpu.VMEM((NBUF, CH, D, b_per_w), jnp.float32),
            pltpu.VMEM((D, b_per_w), jnp.float32),
            [pltpu.SemaphoreType.DMA] * NBUF,
        ],
        compiler_params=pltpu.CompilerParams(
            use_tc_tiling_on_sc=True, needs_layout_passes=False
        ),
    )
    def k(inp_t_hbm, idx_hbm, out_t_hbm, idx_v, slab_v, cols_v, sems):
        wid = lax.axis_index("s") * NC + lax.axis_index("c")
        base = pl.multiple_of(wid * b_per_w, b_per_w)
        pltpu.sync_copy(idx_hbm.at[pl.ds(base, b_per_w)], idx_v)
        vecs = [idx_v[pl.ds(k * L, L)] for k in range(b_per_w // L)]

        def scalar_idx(i):
            return vecs[i // L][i % L]

        def issue(g):
            buf = g % NBUF
            handles = []
            for j in range(CH):
                i = g * CH + j
                n = scalar_idx(i)
                handles.append(
                    pltpu.make_async_copy(
                        inp_t_hbm.at[n, :, pl.ds(base, b_per_w)],
                        slab_v.at[buf, j],
                        sems[buf],
                    )
                )
            for h in handles:
                h.start()
            return handles

        d_vecs = [lax.iota(jnp.int32, L) + k * L for k in range(D // L)]

        def extract(g, handles):
            buf = g % NBUF
            for h in handles:
                h.wait()
            for j in range(CH):
                i = g * CH + j
                jj = jnp.full((L,), j, jnp.int32)
                lane = jnp.full((L,), i, jnp.int32)
                for kk in range(D // L):
                    vals = plsc.load_gather(
                        slab_v.at[buf], [jj, d_vecs[kk], lane]
                    )
                    plsc.store_scatter(cols_v, [d_vecs[kk], lane], vals)

        handles = issue(0)
        for g in range(n_chunks):
            nxt = issue(g + 1) if g + 1 < n_chunks else None
            extract(g, handles)
            handles = nxt
        pltpu.sync_copy(cols_v, out_t_hbm.at[:, pl.ds(base, b_per_w)])

    return k


def kernel(inputs, indexs):
    B, N, D = inputs.shape
    inp_t = jnp.transpose(inputs, (1, 2, 0))
    idx = indexs.reshape(B)
    out_t = _make_gather(B, N, D)(inp_t, idx)
    return jnp.transpose(out_t)
